# baseline (speedup 1.0000x reference)
import functools

import jax
import jax.numpy as jnp
from jax import lax
from jax.experimental import pallas as pl
from jax.experimental.pallas import tpu as pltpu
from jax.experimental.pallas import tpu_sc as plsc

USERS = 100000
ITEMS = 100000
H = 128
B = 4096
L = 200

NC, NS = 2, 16
NW = NC * NS
BPW = B // NW
HROWS = 104
LP = 208
NBLK = LP // 16


@functools.partial(
    pl.kernel,
    out_type=jax.ShapeDtypeStruct((B, LP), jnp.float32),
    mesh=plsc.VectorSubcoreMesh(core_axis_name="c", subcore_axis_name="s"),
    scratch_types=[
        pltpu.VMEM((BPW,), jnp.int32),
        pltpu.VMEM((BPW, 2, HROWS), jnp.int32),
        pltpu.VMEM((BPW, H), jnp.float32),
        pltpu.VMEM((4, HROWS, H), jnp.float32),
        pltpu.VMEM((2, LP), jnp.float32),
        pltpu.SemaphoreType.DMA,
        pltpu.SemaphoreType.DMA,
        pltpu.SemaphoreType.DMA,
        pltpu.SemaphoreType.DMA,
        pltpu.SemaphoreType.DMA,
        pltpu.SemaphoreType.DMA,
        pltpu.SemaphoreType.DMA,
    ],
)
def _mf_sc(uid_hbm, iid_hbm, U_hbm, I_hbm, out_hbm,
           uid_v, iid_v, urows_v, rows_v, outb_v, sem_u, sem_h0, sem_h1,
           sem_h2, sem_h3, sem_o0, sem_o1):
    wid = lax.axis_index("s") * NC + lax.axis_index("c")
    base = wid * BPW
    sems = (sem_h0, sem_h1, sem_h2, sem_h3)
    osems = (sem_o0, sem_o1)

    pltpu.sync_copy(uid_hbm.at[pl.ds(base, BPW)], uid_v)
    pltpu.sync_copy(iid_hbm.at[pl.ds(base, BPW)], iid_v)
    pltpu.async_copy(U_hbm.at[uid_v], urows_v, sem_u).wait()

    def half_copy(b, half, slot):
        return pltpu.make_async_copy(
            I_hbm.at[iid_v.at[b, half]],
            rows_v.at[slot],
            sems[slot])

    def out_copy(b, s):
        return pltpu.make_async_copy(
            outb_v.at[s], out_hbm.at[base + b], osems[s])

    iota16 = lax.broadcasted_iota(jnp.int32, (16,), 0)
    BITREV = tuple(int(format(j, "04b")[::-1], 2) for j in range(16))
    _DN = lax.GatherDimensionNumbers(
        offset_dims=(), collapsed_slice_dims=(0,), start_index_map=(0,))

    def _shuf(a, sft):
        idx = iota16 ^ sft
        return lax.gather(a, idx[:, None], _DN, (1,),
                          mode=lax.GatherScatterMode.PROMISE_IN_BOUNDS)

    def _combine(a, c, sft):
        mask = (iota16 & sft) == 0
        return (jnp.where(mask, a, _shuf(c, sft))
                + jnp.where(mask, _shuf(a, sft), c))

    def compute(b, slot, oslot, blk_lo, blk_hi, roff):
        u = [urows_v[b, pl.ds(k * 16, 16)] for k in range(H // 16)]

        def lblk(i, carry):
            l0 = pl.multiple_of(i * 16, 16)
            vecs = []
            for j in range(16):
                l = jnp.minimum(l0 + (BITREV[j] - roff), HROWS - 1)
                acc = rows_v[slot, l, pl.ds(0, 16)] * u[0]
                for k in range(1, H // 16):
                    acc = acc + rows_v[slot, l, pl.ds(k * 16, 16)] * u[k]
                vecs.append(acc)
            for sft in (8, 4, 2, 1):
                vecs = [_combine(vecs[2 * p], vecs[2 * p + 1], sft)
                        for p in range(len(vecs) // 2)]
            outb_v[oslot, pl.ds(l0, 16)] = vecs[0]
            return carry

        lax.fori_loop(blk_lo, blk_hi, lblk, 0)

    half_copy(0, 0, 0).start()
    half_copy(0, 1, 1).start()
    half_copy(1, 0, 2).start()
    half_copy(1, 1, 3).start()

    def pair(g, carry):
        b0 = 2 * g

        @pl.when(b0 >= 2)
        def _():
            out_copy(b0 - 2, 0).wait()

        half_copy(b0, 0, 0).wait()
        compute(b0, 0, 0, 0, 6, 0)

        @pl.when(b0 + 2 < BPW)
        def _():
            half_copy(b0 + 2, 0, 0).start()

        half_copy(b0, 1, 1).wait()
        compute(b0, 1, 0, 6, NBLK, 96)
        out_copy(b0, 0).start()

        @pl.when(b0 + 2 < BPW)
        def _():
            half_copy(b0 + 2, 1, 1).start()

        @pl.when(b0 >= 2)
        def _():
            out_copy(b0 - 1, 1).wait()

        half_copy(b0 + 1, 0, 2).wait()
        compute(b0 + 1, 2, 1, 0, 6, 0)

        @pl.when(b0 + 3 < BPW)
        def _():
            half_copy(b0 + 3, 0, 2).start()

        half_copy(b0 + 1, 1, 3).wait()
        compute(b0 + 1, 3, 1, 6, NBLK, 96)
        out_copy(b0 + 1, 1).start()

        @pl.when(b0 + 3 < BPW)
        def _():
            half_copy(b0 + 3, 1, 3).start()
        return carry

    lax.fori_loop(0, BPW // 2, pair, 0)
    out_copy(BPW - 2, 0).wait()
    out_copy(BPW - 1, 1).wait()


def kernel(userid_input, itemid_input, U, I):
    uid = userid_input.reshape(B).astype(jnp.int32)
    ids = itemid_input.reshape(B, L).astype(jnp.int32)
    h0 = jnp.pad(ids[:, :96], ((0, 0), (0, HROWS - 96)))
    iid = jnp.stack([h0, ids[:, 96:]], axis=1)
    return _mf_sc(uid, iid, U, I)[:, :L]


# speedup vs baseline: 7.4244x; 7.4244x over previous
import functools

import jax
import jax.numpy as jnp
from jax import lax
from jax.experimental import pallas as pl
from jax.experimental.pallas import tpu as pltpu
from jax.experimental.pallas import tpu_sc as plsc

USERS = 100000
ITEMS = 100000
H = 128
B = 4096
L = 200

NC, NS = 2, 16
NW = NC * NS
BPW = B // NW
HROWS = 100
LP = 208
NBLK = LP // 16


@functools.partial(
    pl.kernel,
    out_type=jax.ShapeDtypeStruct((B, LP), jnp.float32),
    mesh=plsc.VectorSubcoreMesh(core_axis_name="c", subcore_axis_name="s"),
    scratch_types=[
        pltpu.VMEM((BPW,), jnp.int32),
        pltpu.VMEM((BPW, 2, HROWS), jnp.int32),
        pltpu.VMEM((BPW, H), jnp.float32),
        pltpu.VMEM((4, HROWS, H), jnp.float32),
        pltpu.VMEM((2, LP), jnp.float32),
        pltpu.SemaphoreType.DMA,
        pltpu.SemaphoreType.DMA,
        pltpu.SemaphoreType.DMA,
        pltpu.SemaphoreType.DMA,
        pltpu.SemaphoreType.DMA,
        pltpu.SemaphoreType.DMA,
        pltpu.SemaphoreType.DMA,
    ],
)
def _mf_sc(uid_hbm, iid_hbm, U_hbm, I_hbm, out_hbm,
           uid_v, iid_v, urows_v, rows_v, outb_v, sem_u, sem_h0, sem_h1,
           sem_h2, sem_h3, sem_o0, sem_o1):
    wid = lax.axis_index("s") * NC + lax.axis_index("c")
    base = wid * BPW
    sems = (sem_h0, sem_h1, sem_h2, sem_h3)
    osems = (sem_o0, sem_o1)

    pltpu.sync_copy(uid_hbm.at[pl.ds(base, BPW)], uid_v)
    pltpu.sync_copy(iid_hbm.at[pl.ds(base, BPW)], iid_v)
    pltpu.async_copy(U_hbm.at[uid_v], urows_v, sem_u).wait()

    def half_copy(b, half, slot):
        return pltpu.make_async_copy(
            I_hbm.at[iid_v.at[b, half]],
            rows_v.at[slot],
            sems[slot])

    def out_copy(b, s):
        return pltpu.make_async_copy(
            outb_v.at[s], out_hbm.at[base + b], osems[s])

    iota16 = lax.broadcasted_iota(jnp.int32, (16,), 0)
    BITREV = tuple(int(format(j, "04b")[::-1], 2) for j in range(16))
    _DN = lax.GatherDimensionNumbers(
        offset_dims=(), collapsed_slice_dims=(0,), start_index_map=(0,))

    def _shuf(a, sft):
        idx = iota16 ^ sft
        return lax.gather(a, idx[:, None], _DN, (1,),
                          mode=lax.GatherScatterMode.PROMISE_IN_BOUNDS)

    def _combine(a, c, sft):
        mask = (iota16 & sft) == 0
        return (jnp.where(mask, a, _shuf(c, sft))
                + jnp.where(mask, _shuf(a, sft), c))

    def _row_dot(slot, l, u):
        acc = rows_v[slot, l, pl.ds(0, 16)] * u[0]
        for k in range(1, H // 16):
            acc = acc + rows_v[slot, l, pl.ds(k * 16, 16)] * u[k]
        return acc

    def _tree_store(vecs, oslot, l0):
        for sft in (8, 4, 2, 1):
            vecs = [_combine(vecs[2 * p], vecs[2 * p + 1], sft)
                    for p in range(len(vecs) // 2)]
        outb_v[oslot, pl.ds(l0, 16)] = vecs[0]

    def _uload(b):
        return [urows_v[b, pl.ds(k * 16, 16)] for k in range(H // 16)]

    def compute_fori(b, slot, oslot, blk_lo, blk_hi, roff):
        u = _uload(b)

        def lblk(i, carry):
            l0 = pl.multiple_of(i * 16, 16)
            vecs = []
            for j in range(16):
                l = jnp.minimum(l0 + (BITREV[j] - roff), HROWS - 1)
                vecs.append(_row_dot(slot, l, u))
            _tree_store(vecs, oslot, l0)
            return carry

        lax.fori_loop(blk_lo, blk_hi, lblk, 0)

    def compute_straddle(b, slot_lo, slot_hi, oslot):
        u = _uload(b)
        vecs = []
        for j in range(16):
            r = BITREV[j]
            if 96 + r < 2 * HROWS - 100:
                vecs.append(_row_dot(slot_lo, 96 + r, u))
            else:
                vecs.append(_row_dot(slot_hi, r - 4, u))
        _tree_store(vecs, oslot, 96)

    half_copy(0, 0, 0).start()
    half_copy(0, 1, 1).start()
    half_copy(1, 0, 2).start()
    half_copy(1, 1, 3).start()

    def pair(g, carry):
        b0 = 2 * g

        @pl.when(b0 >= 2)
        def _():
            out_copy(b0 - 2, 0).wait()

        half_copy(b0, 0, 0).wait()
        compute_fori(b0, 0, 0, 0, 6, 0)
        half_copy(b0, 1, 1).wait()
        compute_straddle(b0, 0, 1, 0)

        @pl.when(b0 + 2 < BPW)
        def _():
            half_copy(b0 + 2, 0, 0).start()

        compute_fori(b0, 1, 0, 7, NBLK, 100)
        out_copy(b0, 0).start()

        @pl.when(b0 + 2 < BPW)
        def _():
            half_copy(b0 + 2, 1, 1).start()

        @pl.when(b0 >= 2)
        def _():
            out_copy(b0 - 1, 1).wait()

        half_copy(b0 + 1, 0, 2).wait()
        compute_fori(b0 + 1, 2, 1, 0, 6, 0)
        half_copy(b0 + 1, 1, 3).wait()
        compute_straddle(b0 + 1, 2, 3, 1)

        @pl.when(b0 + 3 < BPW)
        def _():
            half_copy(b0 + 3, 0, 2).start()

        compute_fori(b0 + 1, 3, 1, 7, NBLK, 100)
        out_copy(b0 + 1, 1).start()

        @pl.when(b0 + 3 < BPW)
        def _():
            half_copy(b0 + 3, 1, 3).start()
        return carry

    lax.fori_loop(0, BPW // 2, pair, 0)
    out_copy(BPW - 2, 0).wait()
    out_copy(BPW - 1, 1).wait()


def kernel(userid_input, itemid_input, U, I):
    uid = userid_input.reshape(B).astype(jnp.int32)
    iid = itemid_input.reshape(B, 2, HROWS).astype(jnp.int32)
    return _mf_sc(uid, iid, U, I)[:, :L]


# speedup vs baseline: 7.6060x; 1.0245x over previous
import functools

import jax
import jax.numpy as jnp
from jax import lax
from jax.experimental import pallas as pl
from jax.experimental.pallas import tpu as pltpu
from jax.experimental.pallas import tpu_sc as plsc

USERS = 100000
ITEMS = 100000
H = 128
B = 4096
L = 200

NC, NS = 2, 16
NW = NC * NS
BPW = B // NW
HROWS = 100
LP = 208
NBLK = LP // 16


@functools.partial(
    pl.kernel,
    out_type=jax.ShapeDtypeStruct((B, LP), jnp.float32),
    mesh=plsc.VectorSubcoreMesh(core_axis_name="c", subcore_axis_name="s"),
    scratch_types=[
        pltpu.VMEM((BPW,), jnp.int32),
        pltpu.VMEM((BPW, 2, HROWS), jnp.int32),
        pltpu.VMEM((BPW, H), jnp.float32),
        pltpu.VMEM((4, HROWS, H), jnp.float32),
        pltpu.VMEM((2, LP), jnp.float32),
        pltpu.SemaphoreType.DMA,
        pltpu.SemaphoreType.DMA,
        pltpu.SemaphoreType.DMA,
        pltpu.SemaphoreType.DMA,
        pltpu.SemaphoreType.DMA,
        pltpu.SemaphoreType.DMA,
        pltpu.SemaphoreType.DMA,
    ],
)
def _mf_sc(uid_hbm, iid_hbm, U_hbm, I_hbm, out_hbm,
           uid_v, iid_v, urows_v, rows_v, outb_v, sem_u, sem_h0, sem_h1,
           sem_h2, sem_h3, sem_o0, sem_o1):
    wid = lax.axis_index("s") * NC + lax.axis_index("c")
    base = wid * BPW
    sems = (sem_h0, sem_h1, sem_h2, sem_h3)
    osems = (sem_o0, sem_o1)

    pltpu.sync_copy(uid_hbm.at[pl.ds(base, BPW)], uid_v)
    pltpu.sync_copy(iid_hbm.at[pl.ds(base, BPW)], iid_v)
    pltpu.async_copy(U_hbm.at[uid_v], urows_v, sem_u).wait()

    def half_copy(b, half, slot):
        return pltpu.make_async_copy(
            I_hbm.at[iid_v.at[b, half]],
            rows_v.at[slot],
            sems[slot])

    def out_copy(b, s):
        return pltpu.make_async_copy(
            outb_v.at[s], out_hbm.at[base + b], osems[s])

    iota16 = lax.broadcasted_iota(jnp.int32, (16,), 0)
    BITREV = tuple(int(format(j, "04b")[::-1], 2) for j in range(16))
    _DN = lax.GatherDimensionNumbers(
        offset_dims=(), collapsed_slice_dims=(0,), start_index_map=(0,))

    def _shuf(a, sft):
        idx = iota16 ^ sft
        return lax.gather(a, idx[:, None], _DN, (1,),
                          mode=lax.GatherScatterMode.PROMISE_IN_BOUNDS)

    def _combine(a, c, sft):
        mask = (iota16 & sft) == 0
        return (jnp.where(mask, a, _shuf(c, sft))
                + jnp.where(mask, _shuf(a, sft), c))

    def _row_dot(slot, l, u):
        acc = rows_v[slot, l, pl.ds(0, 16)] * u[0]
        for k in range(1, H // 16):
            acc = acc + rows_v[slot, l, pl.ds(k * 16, 16)] * u[k]
        return acc

    def _tree_store(vecs, oslot, l0):
        for sft in (8, 4, 2, 1):
            vecs = [_combine(vecs[2 * p], vecs[2 * p + 1], sft)
                    for p in range(len(vecs) // 2)]
        outb_v[oslot, pl.ds(l0, 16)] = vecs[0]

    def _uload(b):
        return [urows_v[b, pl.ds(k * 16, 16)] for k in range(H // 16)]

    def compute_fori(b, slot, oslot, blk_lo, blk_hi, roff):
        u = _uload(b)

        def lblk(i, carry):
            l0 = pl.multiple_of(i * 16, 16)
            vecs = []
            for j in range(16):
                l = l0 + (BITREV[j] - roff)
                vecs.append(_row_dot(slot, l, u))
            _tree_store(vecs, oslot, l0)
            return carry

        lax.fori_loop(blk_lo, blk_hi, lblk, 0)

    def compute_straddle(b, slot_lo, slot_hi, oslot):
        u = _uload(b)
        vecs = []
        for j in range(16):
            r = BITREV[j]
            if 96 + r < 2 * HROWS - 100:
                vecs.append(_row_dot(slot_lo, 96 + r, u))
            else:
                vecs.append(_row_dot(slot_hi, r - 4, u))
        _tree_store(vecs, oslot, 96)

    def compute_tail(b, slot, oslot):
        u = _uload(b)
        zero = jnp.zeros((16,), jnp.float32)
        vecs = []
        for j in range(16):
            r = BITREV[j]
            if r < 8:
                vecs.append(_row_dot(slot, 92 + r, u))
            else:
                vecs.append(zero)
        _tree_store(vecs, oslot, 192)

    half_copy(0, 0, 0).start()
    half_copy(0, 1, 1).start()
    half_copy(1, 0, 2).start()
    half_copy(1, 1, 3).start()

    def pair(g, carry):
        b0 = 2 * g

        @pl.when(b0 >= 2)
        def _():
            out_copy(b0 - 2, 0).wait()

        half_copy(b0, 0, 0).wait()
        compute_fori(b0, 0, 0, 0, 6, 0)
        half_copy(b0, 1, 1).wait()
        compute_straddle(b0, 0, 1, 0)

        @pl.when(b0 + 2 < BPW)
        def _():
            half_copy(b0 + 2, 0, 0).start()

        compute_fori(b0, 1, 0, 7, NBLK - 1, 100)
        compute_tail(b0, 1, 0)
        out_copy(b0, 0).start()

        @pl.when(b0 + 2 < BPW)
        def _():
            half_copy(b0 + 2, 1, 1).start()

        @pl.when(b0 >= 2)
        def _():
            out_copy(b0 - 1, 1).wait()

        half_copy(b0 + 1, 0, 2).wait()
        compute_fori(b0 + 1, 2, 1, 0, 6, 0)
        half_copy(b0 + 1, 1, 3).wait()
        compute_straddle(b0 + 1, 2, 3, 1)

        @pl.when(b0 + 3 < BPW)
        def _():
            half_copy(b0 + 3, 0, 2).start()

        compute_fori(b0 + 1, 3, 1, 7, NBLK - 1, 100)
        compute_tail(b0 + 1, 3, 1)
        out_copy(b0 + 1, 1).start()

        @pl.when(b0 + 3 < BPW)
        def _():
            half_copy(b0 + 3, 1, 3).start()
        return carry

    lax.fori_loop(0, BPW // 2, pair, 0)
    out_copy(BPW - 2, 0).wait()
    out_copy(BPW - 1, 1).wait()


def kernel(userid_input, itemid_input, U, I):
    uid = userid_input.reshape(B).astype(jnp.int32)
    iid = itemid_input.reshape(B, 2, HROWS).astype(jnp.int32)
    return _mf_sc(uid, iid, U, I)[:, :L]


# speedup vs baseline: 7.6162x; 1.0013x over previous
import functools

import jax
import jax.numpy as jnp
from jax import lax
from jax.experimental import pallas as pl
from jax.experimental.pallas import tpu as pltpu
from jax.experimental.pallas import tpu_sc as plsc

USERS = 100000
ITEMS = 100000
H = 128
B = 4096
L = 200

NC, NS = 2, 16
NW = NC * NS
BPW = B // NW
HROWS = 100
LP = 208
NBLK = LP // 16


@functools.partial(
    pl.kernel,
    out_type=jax.ShapeDtypeStruct((B, LP), jnp.float32),
    mesh=plsc.VectorSubcoreMesh(core_axis_name="c", subcore_axis_name="s"),
    scratch_types=[
        pltpu.VMEM((BPW,), jnp.int32),
        pltpu.VMEM((BPW, 2, HROWS), jnp.int32),
        pltpu.VMEM((BPW, H), jnp.float32),
        pltpu.VMEM((4, HROWS, H), jnp.float32),
        pltpu.VMEM((2, LP), jnp.float32),
        pltpu.SemaphoreType.DMA,
        pltpu.SemaphoreType.DMA,
        pltpu.SemaphoreType.DMA,
        pltpu.SemaphoreType.DMA,
        pltpu.SemaphoreType.DMA,
        pltpu.SemaphoreType.DMA,
        pltpu.SemaphoreType.DMA,
    ],
)
def _mf_sc(uid_hbm, iid_hbm, U_hbm, I_hbm, out_hbm,
           uid_v, iid_v, urows_v, rows_v, outb_v, sem_u, sem_h0, sem_h1,
           sem_h2, sem_h3, sem_o0, sem_o1):
    wid = lax.axis_index("s") * NC + lax.axis_index("c")
    base = wid * BPW
    sems = (sem_h0, sem_h1, sem_h2, sem_h3)
    osems = (sem_o0, sem_o1)

    pltpu.sync_copy(uid_hbm.at[pl.ds(base, BPW)], uid_v)
    pltpu.sync_copy(iid_hbm.at[pl.ds(base, BPW)], iid_v)
    pltpu.async_copy(U_hbm.at[uid_v], urows_v, sem_u).wait()

    def half_copy(b, half, slot):
        return pltpu.make_async_copy(
            I_hbm.at[iid_v.at[b, half]],
            rows_v.at[slot],
            sems[slot])

    def out_copy(b, s):
        return pltpu.make_async_copy(
            outb_v.at[s], out_hbm.at[base + b], osems[s])

    iota16 = lax.broadcasted_iota(jnp.int32, (16,), 0)
    BITREV = tuple(int(format(j, "04b")[::-1], 2) for j in range(16))
    _DN = lax.GatherDimensionNumbers(
        offset_dims=(), collapsed_slice_dims=(0,), start_index_map=(0,))

    def _shuf(a, sft):
        idx = iota16 ^ sft
        return lax.gather(a, idx[:, None], _DN, (1,),
                          mode=lax.GatherScatterMode.PROMISE_IN_BOUNDS)

    def _combine(a, c, sft):
        mask = (iota16 & sft) == 0
        return (jnp.where(mask, a, _shuf(c, sft))
                + jnp.where(mask, _shuf(a, sft), c))

    def _row_dot(slot, l, u):
        acc = rows_v[slot, l, pl.ds(0, 16)] * u[0]
        for k in range(1, H // 16):
            acc = acc + rows_v[slot, l, pl.ds(k * 16, 16)] * u[k]
        return acc

    def _tree_store(vecs, oslot, l0):
        for sft in (8, 4, 2, 1):
            vecs = [_combine(vecs[2 * p], vecs[2 * p + 1], sft)
                    for p in range(len(vecs) // 2)]
        outb_v[oslot, pl.ds(l0, 16)] = vecs[0]

    def _uload(b):
        return [urows_v[b, pl.ds(k * 16, 16)] for k in range(H // 16)]

    def compute_fori(b, slot, oslot, blk_lo, blk_hi, roff):
        u = _uload(b)

        def lblk(i, carry):
            l0 = pl.multiple_of(i * 16, 16)
            vecs = []
            for j in range(16):
                l = l0 + (BITREV[j] - roff)
                vecs.append(_row_dot(slot, l, u))
            _tree_store(vecs, oslot, l0)
            return carry

        lax.fori_loop(blk_lo, blk_hi, lblk, 0)

    def compute_straddle(b, slot_lo, slot_hi, oslot):
        u = _uload(b)
        vecs = []
        for j in range(16):
            r = BITREV[j]
            if r < 4:
                vecs.append(_row_dot(slot_lo, 96 + r, u))
            else:
                vecs.append(_row_dot(slot_hi, r - 4, u))
        _tree_store(vecs, oslot, 96)

    def compute_tail(b, slot, oslot):
        u = _uload(b)
        zero = jnp.zeros((16,), jnp.float32)
        vecs = []
        for j in range(16):
            r = BITREV[j]
            if r < 8:
                vecs.append(_row_dot(slot, 92 + r, u))
            else:
                vecs.append(zero)
        _tree_store(vecs, oslot, 192)

    half_copy(0, 0, 0).start()
    half_copy(0, 1, 1).start()
    half_copy(1, 0, 2).start()
    half_copy(1, 1, 3).start()

    def pair(g, carry):
        b0 = 2 * g

        @pl.when(b0 >= 2)
        def _():
            out_copy(b0 - 2, 0).wait()

        half_copy(b0, 0, 0).wait()
        compute_fori(b0, 0, 0, 0, 6, 0)
        half_copy(b0, 1, 1).wait()
        compute_straddle(b0, 0, 1, 0)

        @pl.when(b0 + 2 < BPW)
        def _():
            half_copy(b0 + 2, 0, 0).start()

        compute_fori(b0, 1, 0, 7, NBLK - 1, 100)
        compute_tail(b0, 1, 0)
        out_copy(b0, 0).start()

        @pl.when(b0 + 2 < BPW)
        def _():
            half_copy(b0 + 2, 1, 1).start()

        @pl.when(b0 >= 2)
        def _():
            out_copy(b0 - 1, 1).wait()

        half_copy(b0 + 1, 0, 2).wait()
        compute_fori(b0 + 1, 2, 1, 0, 6, 0)
        half_copy(b0 + 1, 1, 3).wait()
        compute_straddle(b0 + 1, 2, 3, 1)

        @pl.when(b0 + 3 < BPW)
        def _():
            half_copy(b0 + 3, 0, 2).start()

        compute_fori(b0 + 1, 3, 1, 7, NBLK - 1, 100)
        compute_tail(b0 + 1, 3, 1)
        out_copy(b0 + 1, 1).start()

        @pl.when(b0 + 3 < BPW)
        def _():
            half_copy(b0 + 3, 1, 3).start()
        return carry

    lax.fori_loop(0, BPW // 2, pair, 0)
    out_copy(BPW - 2, 0).wait()
    out_copy(BPW - 1, 1).wait()


def kernel(userid_input, itemid_input, U, I):
    uid = userid_input.reshape(B).astype(jnp.int32)
    iid = itemid_input.reshape(B, 2, HROWS).astype(jnp.int32)
    return _mf_sc(uid, iid, U, I)[:, :L]
